# baseline (speedup 1.0000x reference)
import functools

import jax
import jax.numpy as jnp
from jax import lax
from jax.experimental import pallas as pl
from jax.experimental.pallas import tpu as pltpu
from jax.experimental.pallas import tpu_sc as plsc

IN_DIM = 1000000
HID = 32
BATCH = 16384
HIST = 50

NC = 2
NS = 16
NW = NC * NS
LANES = 16

SUPER = 128 // HID
RBLK = 512
NTBLK = (IN_DIM + RBLK - 1) // RBLK
NSUPER = NTBLK * (RBLK // SUPER)
ROWS_PER_W = BATCH // NW
CHUNK = 8
IDX_PER_CHUNK = CHUNK * HIST
SUB = 5
IDX_PER_SUB = IDX_PER_CHUNK // SUB
CHUNKS_PER_BLK = 16
IDX_PER_BLK = IDX_PER_CHUNK * CHUNKS_PER_BLK
NBLK = ROWS_PER_W // (CHUNK * CHUNKS_PER_BLK)
NACC = 4
NGRP = (HIST + LANES - 1) // LANES


def _encoder_body(inp_hbm, lut_hbm, bias_hbm, out_hbm,
                  idxblk_v, q0_v, q1_v, rows0_v, rows1_v, out_v, bias_v,
                  sem0, sem1):
    wid = lax.axis_index("c") * NS + lax.axis_index("s")
    wrow0 = wid * ROWS_PER_W
    widx0 = wrow0 * HIST

    pltpu.sync_copy(bias_hbm, bias_v)
    bias_lo = bias_v[pl.ds(0, LANES)]
    bias_hi = bias_v[pl.ds(LANES, LANES)]

    def stage(cc, q_ref, rows_ref, sem):
        off = cc * IDX_PER_CHUNK

        def qbody(i, carry):
            v = idxblk_v[pl.ds(off + i * LANES, LANES)]
            q_ref[pl.ds(i * LANES, LANES)] = (
                (v & 127)
                + lax.shift_left(lax.shift_right_logical(v, 9), 7))
            return carry

        lax.fori_loop(0, IDX_PER_CHUNK // LANES, qbody, 0)
        for j in range(SUB):
            pltpu.async_copy(
                lut_hbm.at[q_ref.at[pl.ds(j * IDX_PER_SUB, IDX_PER_SUB)]],
                rows_ref.at[pl.ds(j * IDX_PER_SUB, IDX_PER_SUB)],
                sem)

    def consume(parity, cc, rows_ref, sem):
        pltpu.make_async_copy(lut_hbm.at[pl.ds(0, IDX_PER_CHUNK)],
                              rows_ref, sem).wait()

        def body(b, carry):
            ibase = cc * IDX_PER_CHUNK + b * HIST
            acc = [None] * (2 * NACC)
            for g in range(NGRP):
                n = min(LANES, HIST - g * LANES)
                iv = idxblk_v[pl.ds(ibase + g * LANES, LANES)]
                rv = (lax.shift_right_logical(iv, 7) & 3) * HID
                for t in range(n):
                    lane_off = rv[t]
                    row = b * HIST + g * LANES + t
                    l = g * LANES + t
                    k = l % NACC
                    lo = rows_ref[row, pl.ds(lane_off, LANES)]
                    hi = rows_ref[row, pl.ds(lane_off + LANES, LANES)]
                    acc[k] = lo if acc[k] is None else acc[k] + lo
                    kh = NACC + k
                    acc[kh] = hi if acc[kh] is None else acc[kh] + hi
            lo_sum = (acc[0] + acc[1]) + (acc[2] + acc[3]) + bias_lo
            hi_sum = (acc[4] + acc[5]) + (acc[6] + acc[7]) + bias_hi
            orow = parity * CHUNK + b
            out_v[orow, pl.ds(0, LANES)] = lo_sum
            out_v[orow, pl.ds(LANES, LANES)] = hi_sum
            return carry

        lax.fori_loop(0, CHUNK, body, 0)

    for blk in range(NBLK):
        pltpu.sync_copy(
            inp_hbm.at[pl.ds(widx0 + blk * IDX_PER_BLK, IDX_PER_BLK)],
            idxblk_v.at[pl.ds(0, IDX_PER_BLK)])
        stage(0, q0_v, rows0_v, sem0)

        def pair_body(p, carry, blk=blk):
            c0 = 2 * p
            stage(c0 + 1, q1_v, rows1_v, sem1)
            consume(0, c0, rows0_v, sem0)

            @pl.when(c0 + 2 < CHUNKS_PER_BLK)
            def _():
                stage(c0 + 2, q0_v, rows0_v, sem0)

            consume(1, c0 + 1, rows1_v, sem1)
            pltpu.sync_copy(
                out_v,
                out_hbm.at[pl.ds(wrow0 + blk * (CHUNKS_PER_BLK * CHUNK)
                                 + c0 * CHUNK, 2 * CHUNK)])
            return carry

        lax.fori_loop(0, CHUNKS_PER_BLK // 2, pair_body, 0)


_encoder = functools.partial(
    pl.kernel,
    out_type=jax.ShapeDtypeStruct((BATCH, HID), jnp.float32),
    mesh=plsc.VectorSubcoreMesh(core_axis_name="c", subcore_axis_name="s",
                                num_cores=NC, num_subcores=NS),
    scratch_types=[
        pltpu.VMEM((IDX_PER_BLK + 2 * LANES,), jnp.int32),
        pltpu.VMEM((IDX_PER_CHUNK,), jnp.int32),
        pltpu.VMEM((IDX_PER_CHUNK,), jnp.int32),
        pltpu.VMEM((IDX_PER_CHUNK, 128), jnp.float32),
        pltpu.VMEM((IDX_PER_CHUNK, 128), jnp.float32),
        pltpu.VMEM((2 * CHUNK, HID), jnp.float32),
        pltpu.VMEM((HID,), jnp.float32),
        pltpu.SemaphoreType.DMA,
        pltpu.SemaphoreType.DMA,
    ],
    compiler_params=pltpu.CompilerParams(use_tc_tiling_on_sc=True),
)(_encoder_body)


def _transpose_body(lut_t_ref, out_ref):
    for u in range(SUPER):
        x = lut_t_ref[:, pl.ds(u * 128, 128)]
        out_ref[:, pl.ds(u * HID, HID)] = jnp.transpose(x)


_transpose_tc = pl.pallas_call(
    _transpose_body,
    out_shape=jax.ShapeDtypeStruct((NSUPER, SUPER * HID), jnp.float32),
    grid=(NTBLK,),
    in_specs=[pl.BlockSpec((HID, RBLK), lambda j: (0, j))],
    out_specs=pl.BlockSpec((RBLK // SUPER, SUPER * HID), lambda j: (j, 0)),
)


def kernel(inp, lut, bias):
    inp_flat = inp.reshape(-1).astype(jnp.int32)
    lut_wide = _transpose_tc(lut.T)
    return _encoder(inp_flat, lut_wide, bias)


# speedup vs baseline: 2.5377x; 2.5377x over previous
import functools

import jax
import jax.numpy as jnp
from jax import lax
from jax.experimental import pallas as pl
from jax.experimental.pallas import tpu as pltpu
from jax.experimental.pallas import tpu_sc as plsc

IN_DIM = 1000000
HID = 32
BATCH = 16384
HIST = 50

NC = 2
NS = 16
NW = NC * NS
LANES = 16

SUPER = 128 // HID
RBLK = 512
TCW = 4
TCBLK = RBLK * TCW
NTSTEP = (IN_DIM + TCBLK - 1) // TCBLK
NSUPER = NTSTEP * TCW * (RBLK // SUPER)
ROWS_PER_W = BATCH // NW
CHUNK = 8
IDX_PER_CHUNK = CHUNK * HIST
SUB = 5
IDX_PER_SUB = IDX_PER_CHUNK // SUB
CHUNKS_PER_BLK = 16
IDX_PER_BLK = IDX_PER_CHUNK * CHUNKS_PER_BLK
NBLK = ROWS_PER_W // (CHUNK * CHUNKS_PER_BLK)
NACC = 4
NGRP = (HIST + LANES - 1) // LANES


def _encoder_body(inp_hbm, lut_hbm, bias_hbm, out_hbm,
                  idxblk_v, q0_v, q1_v, rows0_v, rows1_v, out_v, bias_v,
                  sem0, sem1):
    wid = lax.axis_index("c") * NS + lax.axis_index("s")
    wrow0 = wid * ROWS_PER_W
    widx0 = wrow0 * HIST

    pltpu.sync_copy(bias_hbm, bias_v)
    bias_lo = bias_v[pl.ds(0, LANES)]
    bias_hi = bias_v[pl.ds(LANES, LANES)]

    def stage(cc, q_ref, rows_ref, sem):
        off = cc * IDX_PER_CHUNK

        def qbody(i, carry):
            v = idxblk_v[pl.ds(off + i * LANES, LANES)]
            q_ref[pl.ds(i * LANES, LANES)] = (
                (v & 127)
                + lax.shift_left(lax.shift_right_logical(v, 9), 7))
            return carry

        lax.fori_loop(0, IDX_PER_CHUNK // LANES, qbody, 0)
        for j in range(SUB):
            pltpu.async_copy(
                lut_hbm.at[q_ref.at[pl.ds(j * IDX_PER_SUB, IDX_PER_SUB)]],
                rows_ref.at[pl.ds(j * IDX_PER_SUB, IDX_PER_SUB)],
                sem)

    def consume(parity, cc, rows_ref, sem):
        pltpu.make_async_copy(lut_hbm.at[pl.ds(0, IDX_PER_CHUNK)],
                              rows_ref, sem).wait()

        def body(b, carry):
            ibase = cc * IDX_PER_CHUNK + b * HIST
            acc = [None] * (2 * NACC)
            for g in range(NGRP):
                n = min(LANES, HIST - g * LANES)
                iv = idxblk_v[pl.ds(ibase + g * LANES, LANES)]
                rv = (lax.shift_right_logical(iv, 7) & 3) * HID
                for t in range(n):
                    lane_off = rv[t]
                    row = b * HIST + g * LANES + t
                    l = g * LANES + t
                    k = l % NACC
                    lo = rows_ref[row, pl.ds(lane_off, LANES)]
                    hi = rows_ref[row, pl.ds(lane_off + LANES, LANES)]
                    acc[k] = lo if acc[k] is None else acc[k] + lo
                    kh = NACC + k
                    acc[kh] = hi if acc[kh] is None else acc[kh] + hi
            lo_sum = (acc[0] + acc[1]) + (acc[2] + acc[3]) + bias_lo
            hi_sum = (acc[4] + acc[5]) + (acc[6] + acc[7]) + bias_hi
            orow = parity * CHUNK + b
            out_v[orow, pl.ds(0, LANES)] = lo_sum
            out_v[orow, pl.ds(LANES, LANES)] = hi_sum
            return carry

        lax.fori_loop(0, CHUNK, body, 0)

    for blk in range(NBLK):
        pltpu.sync_copy(
            inp_hbm.at[pl.ds(widx0 + blk * IDX_PER_BLK, IDX_PER_BLK)],
            idxblk_v.at[pl.ds(0, IDX_PER_BLK)])
        stage(0, q0_v, rows0_v, sem0)

        def pair_body(p, carry, blk=blk):
            c0 = 2 * p
            stage(c0 + 1, q1_v, rows1_v, sem1)
            consume(0, c0, rows0_v, sem0)

            @pl.when(c0 + 2 < CHUNKS_PER_BLK)
            def _():
                stage(c0 + 2, q0_v, rows0_v, sem0)

            consume(1, c0 + 1, rows1_v, sem1)
            pltpu.sync_copy(
                out_v,
                out_hbm.at[pl.ds(wrow0 + blk * (CHUNKS_PER_BLK * CHUNK)
                                 + c0 * CHUNK, 2 * CHUNK)])
            return carry

        lax.fori_loop(0, CHUNKS_PER_BLK // 2, pair_body, 0)


_encoder = functools.partial(
    pl.kernel,
    out_type=jax.ShapeDtypeStruct((BATCH, HID), jnp.float32),
    mesh=plsc.VectorSubcoreMesh(core_axis_name="c", subcore_axis_name="s",
                                num_cores=NC, num_subcores=NS),
    scratch_types=[
        pltpu.VMEM((IDX_PER_BLK + 2 * LANES,), jnp.int32),
        pltpu.VMEM((IDX_PER_CHUNK,), jnp.int32),
        pltpu.VMEM((IDX_PER_CHUNK,), jnp.int32),
        pltpu.VMEM((IDX_PER_CHUNK, 128), jnp.float32),
        pltpu.VMEM((IDX_PER_CHUNK, 128), jnp.float32),
        pltpu.VMEM((2 * CHUNK, HID), jnp.float32),
        pltpu.VMEM((HID,), jnp.float32),
        pltpu.SemaphoreType.DMA,
        pltpu.SemaphoreType.DMA,
    ],
    compiler_params=pltpu.CompilerParams(use_tc_tiling_on_sc=True),
)(_encoder_body)


def _transpose_body(lut_t_ref, out_ref):
    for w in range(TCW):
        x = jnp.concatenate(
            [lut_t_ref[:, pl.ds(w * RBLK + u * 128, 128)]
             for u in range(SUPER)], axis=0)
        out_ref[pl.ds(w * 128, 128), :] = jnp.transpose(x)


_transpose_tc = pl.pallas_call(
    _transpose_body,
    out_shape=jax.ShapeDtypeStruct((NSUPER, SUPER * HID), jnp.float32),
    grid=(NTSTEP,),
    in_specs=[pl.BlockSpec((HID, TCBLK), lambda j: (0, j))],
    out_specs=pl.BlockSpec((TCBLK // SUPER, SUPER * HID), lambda j: (j, 0)),
)


def kernel(inp, lut, bias):
    inp_flat = inp.reshape(-1).astype(jnp.int32)
    lut_wide = _transpose_tc(lut.T)
    return _encoder(inp_flat, lut_wide, bias)


# speedup vs baseline: 3.2712x; 1.2890x over previous
import functools

import jax
import jax.numpy as jnp
from jax import lax
from jax.experimental import pallas as pl
from jax.experimental.pallas import tpu as pltpu
from jax.experimental.pallas import tpu_sc as plsc

IN_DIM = 1000000
HID = 32
BATCH = 16384
HIST = 50

NC = 2
NS = 16
NW = NC * NS
LANES = 16

SUPER = 128 // HID
RBLK = 512
TCW = 8
TCBLK = RBLK * TCW
NTSTEP = (IN_DIM + TCBLK - 1) // TCBLK
NSUPER = NTSTEP * TCW * (RBLK // SUPER)
ROWS_PER_W = BATCH // NW
CHUNK = 8
IDX_PER_CHUNK = CHUNK * HIST
SUB = 5
IDX_PER_SUB = IDX_PER_CHUNK // SUB
CHUNKS_PER_BLK = 16
IDX_PER_BLK = IDX_PER_CHUNK * CHUNKS_PER_BLK
NBLK = ROWS_PER_W // (CHUNK * CHUNKS_PER_BLK)
NACC = 4
NGRP = (HIST + LANES - 1) // LANES


def _encoder_body(inp_hbm, lut_hbm, bias_hbm, out_hbm,
                  idxblk_v, q0_v, q1_v, rows0_v, rows1_v, out_v, bias_v,
                  sem0, sem1):
    wid = lax.axis_index("c") * NS + lax.axis_index("s")
    wrow0 = wid * ROWS_PER_W
    widx0 = wrow0 * HIST

    pltpu.sync_copy(bias_hbm, bias_v)
    bias_lo = bias_v[pl.ds(0, LANES)]
    bias_hi = bias_v[pl.ds(LANES, LANES)]

    def stage(cc, q_ref, rows_ref, sem):
        off = cc * IDX_PER_CHUNK

        def qbody(i, carry):
            v = idxblk_v[pl.ds(off + i * LANES, LANES)]
            q_ref[pl.ds(i * LANES, LANES)] = (
                (v & 127)
                + lax.shift_left(lax.shift_right_logical(v, 9), 7))
            return carry

        lax.fori_loop(0, IDX_PER_CHUNK // LANES, qbody, 0)
        for j in range(SUB):
            pltpu.async_copy(
                lut_hbm.at[q_ref.at[pl.ds(j * IDX_PER_SUB, IDX_PER_SUB)]],
                rows_ref.at[pl.ds(j * IDX_PER_SUB, IDX_PER_SUB)],
                sem)

    def consume(parity, cc, rows_ref, sem):
        pltpu.make_async_copy(lut_hbm.at[pl.ds(0, IDX_PER_CHUNK)],
                              rows_ref, sem).wait()

        def body(b, carry):
            ibase = cc * IDX_PER_CHUNK + b * HIST
            acc = [None] * (2 * NACC)
            for g in range(NGRP):
                n = min(LANES, HIST - g * LANES)
                iv = idxblk_v[pl.ds(ibase + g * LANES, LANES)]
                rv = (lax.shift_right_logical(iv, 7) & 3) * HID
                for t in range(n):
                    lane_off = rv[t]
                    row = b * HIST + g * LANES + t
                    l = g * LANES + t
                    k = l % NACC
                    lo = rows_ref[row, pl.ds(lane_off, LANES)]
                    hi = rows_ref[row, pl.ds(lane_off + LANES, LANES)]
                    acc[k] = lo if acc[k] is None else acc[k] + lo
                    kh = NACC + k
                    acc[kh] = hi if acc[kh] is None else acc[kh] + hi
            lo_sum = (acc[0] + acc[1]) + (acc[2] + acc[3]) + bias_lo
            hi_sum = (acc[4] + acc[5]) + (acc[6] + acc[7]) + bias_hi
            orow = parity * CHUNK + b
            out_v[orow, pl.ds(0, LANES)] = lo_sum
            out_v[orow, pl.ds(LANES, LANES)] = hi_sum
            return carry

        lax.fori_loop(0, CHUNK, body, 0)

    for blk in range(NBLK):
        pltpu.sync_copy(
            inp_hbm.at[pl.ds(widx0 + blk * IDX_PER_BLK, IDX_PER_BLK)],
            idxblk_v.at[pl.ds(0, IDX_PER_BLK)])
        stage(0, q0_v, rows0_v, sem0)

        def pair_body(p, carry, blk=blk):
            c0 = 2 * p
            stage(c0 + 1, q1_v, rows1_v, sem1)
            consume(0, c0, rows0_v, sem0)

            @pl.when(c0 + 2 < CHUNKS_PER_BLK)
            def _():
                stage(c0 + 2, q0_v, rows0_v, sem0)

            consume(1, c0 + 1, rows1_v, sem1)
            pltpu.sync_copy(
                out_v,
                out_hbm.at[pl.ds(wrow0 + blk * (CHUNKS_PER_BLK * CHUNK)
                                 + c0 * CHUNK, 2 * CHUNK)])
            return carry

        lax.fori_loop(0, CHUNKS_PER_BLK // 2, pair_body, 0)


_encoder = functools.partial(
    pl.kernel,
    out_type=jax.ShapeDtypeStruct((BATCH, HID), jnp.float32),
    mesh=plsc.VectorSubcoreMesh(core_axis_name="c", subcore_axis_name="s",
                                num_cores=NC, num_subcores=NS),
    scratch_types=[
        pltpu.VMEM((IDX_PER_BLK + 2 * LANES,), jnp.int32),
        pltpu.VMEM((IDX_PER_CHUNK,), jnp.int32),
        pltpu.VMEM((IDX_PER_CHUNK,), jnp.int32),
        pltpu.VMEM((IDX_PER_CHUNK, 128), jnp.float32),
        pltpu.VMEM((IDX_PER_CHUNK, 128), jnp.float32),
        pltpu.VMEM((2 * CHUNK, HID), jnp.float32),
        pltpu.VMEM((HID,), jnp.float32),
        pltpu.SemaphoreType.DMA,
        pltpu.SemaphoreType.DMA,
    ],
    compiler_params=pltpu.CompilerParams(use_tc_tiling_on_sc=True),
)(_encoder_body)


def _transpose_body(lut_t_ref, out_ref):
    for w in range(TCW):
        x = jnp.concatenate(
            [lut_t_ref[:, pl.ds(w * RBLK + u * 128, 128)]
             for u in range(SUPER)], axis=0)
        out_ref[pl.ds(w * 128, 128), :] = jnp.transpose(x)


_transpose_tc = pl.pallas_call(
    _transpose_body,
    out_shape=jax.ShapeDtypeStruct((NSUPER, SUPER * HID), jnp.float32),
    grid=(NTSTEP,),
    in_specs=[pl.BlockSpec((HID, TCBLK), lambda j: (0, j))],
    out_specs=pl.BlockSpec((TCBLK // SUPER, SUPER * HID), lambda j: (j, 0)),
)


def kernel(inp, lut, bias):
    inp_flat = inp.reshape(-1).astype(jnp.int32)
    lut_wide = _transpose_tc(lut.T)
    return _encoder(inp_flat, lut_wide, bias)


# speedup vs baseline: 3.8215x; 1.1682x over previous
import functools

import jax
import jax.numpy as jnp
from jax import lax
from jax.experimental import pallas as pl
from jax.experimental.pallas import tpu as pltpu
from jax.experimental.pallas import tpu_sc as plsc

IN_DIM = 1000000
HID = 32
BATCH = 16384
HIST = 50

NC = 2
NS = 16
NW = NC * NS
LANES = 16

SUPER = 128 // HID
RBLK = 512
TCW = 16
TCBLK = RBLK * TCW
NTSTEP = (IN_DIM + TCBLK - 1) // TCBLK
NSUPER = NTSTEP * TCW * (RBLK // SUPER)
ROWS_PER_W = BATCH // NW
CHUNK = 8
IDX_PER_CHUNK = CHUNK * HIST
SUB = 5
IDX_PER_SUB = IDX_PER_CHUNK // SUB
CHUNKS_PER_BLK = 16
IDX_PER_BLK = IDX_PER_CHUNK * CHUNKS_PER_BLK
NBLK = ROWS_PER_W // (CHUNK * CHUNKS_PER_BLK)
NACC = 4
NGRP = (HIST + LANES - 1) // LANES


def _encoder_body(inp_hbm, lut_hbm, bias_hbm, out_hbm,
                  idxblk_v, q0_v, q1_v, rows0_v, rows1_v, out_v, bias_v,
                  sem0, sem1):
    wid = lax.axis_index("c") * NS + lax.axis_index("s")
    wrow0 = wid * ROWS_PER_W
    widx0 = wrow0 * HIST

    pltpu.sync_copy(bias_hbm, bias_v)
    bias_lo = bias_v[pl.ds(0, LANES)]
    bias_hi = bias_v[pl.ds(LANES, LANES)]

    def stage(cc, q_ref, rows_ref, sem):
        off = cc * IDX_PER_CHUNK

        def qbody(i, carry):
            v = idxblk_v[pl.ds(off + i * LANES, LANES)]
            q_ref[pl.ds(i * LANES, LANES)] = (
                (v & 127)
                + lax.shift_left(lax.shift_right_logical(v, 9), 7))
            return carry

        lax.fori_loop(0, IDX_PER_CHUNK // LANES, qbody, 0)
        for j in range(SUB):
            pltpu.async_copy(
                lut_hbm.at[q_ref.at[pl.ds(j * IDX_PER_SUB, IDX_PER_SUB)]],
                rows_ref.at[pl.ds(j * IDX_PER_SUB, IDX_PER_SUB)],
                sem)

    def consume(parity, cc, rows_ref, sem):
        pltpu.make_async_copy(lut_hbm.at[pl.ds(0, IDX_PER_CHUNK)],
                              rows_ref, sem).wait()

        def body(b, carry):
            ibase = cc * IDX_PER_CHUNK + b * HIST
            acc = [None] * (2 * NACC)
            for g in range(NGRP):
                n = min(LANES, HIST - g * LANES)
                iv = idxblk_v[pl.ds(ibase + g * LANES, LANES)]
                rv = (lax.shift_right_logical(iv, 7) & 3) * HID
                for t in range(n):
                    lane_off = rv[t]
                    row = b * HIST + g * LANES + t
                    l = g * LANES + t
                    k = l % NACC
                    lo = rows_ref[row, pl.ds(lane_off, LANES)]
                    hi = rows_ref[row, pl.ds(lane_off + LANES, LANES)]
                    acc[k] = lo if acc[k] is None else acc[k] + lo
                    kh = NACC + k
                    acc[kh] = hi if acc[kh] is None else acc[kh] + hi
            lo_sum = (acc[0] + acc[1]) + (acc[2] + acc[3]) + bias_lo
            hi_sum = (acc[4] + acc[5]) + (acc[6] + acc[7]) + bias_hi
            orow = parity * CHUNK + b
            out_v[orow, pl.ds(0, LANES)] = lo_sum
            out_v[orow, pl.ds(LANES, LANES)] = hi_sum
            return carry

        lax.fori_loop(0, CHUNK, body, 0)

    for blk in range(NBLK):
        pltpu.sync_copy(
            inp_hbm.at[pl.ds(widx0 + blk * IDX_PER_BLK, IDX_PER_BLK)],
            idxblk_v.at[pl.ds(0, IDX_PER_BLK)])
        stage(0, q0_v, rows0_v, sem0)

        def pair_body(p, carry, blk=blk):
            c0 = 2 * p
            stage(c0 + 1, q1_v, rows1_v, sem1)
            consume(0, c0, rows0_v, sem0)

            @pl.when(c0 + 2 < CHUNKS_PER_BLK)
            def _():
                stage(c0 + 2, q0_v, rows0_v, sem0)

            consume(1, c0 + 1, rows1_v, sem1)
            pltpu.sync_copy(
                out_v,
                out_hbm.at[pl.ds(wrow0 + blk * (CHUNKS_PER_BLK * CHUNK)
                                 + c0 * CHUNK, 2 * CHUNK)])
            return carry

        lax.fori_loop(0, CHUNKS_PER_BLK // 2, pair_body, 0)


_encoder = functools.partial(
    pl.kernel,
    out_type=jax.ShapeDtypeStruct((BATCH, HID), jnp.float32),
    mesh=plsc.VectorSubcoreMesh(core_axis_name="c", subcore_axis_name="s",
                                num_cores=NC, num_subcores=NS),
    scratch_types=[
        pltpu.VMEM((IDX_PER_BLK + 2 * LANES,), jnp.int32),
        pltpu.VMEM((IDX_PER_CHUNK,), jnp.int32),
        pltpu.VMEM((IDX_PER_CHUNK,), jnp.int32),
        pltpu.VMEM((IDX_PER_CHUNK, 128), jnp.float32),
        pltpu.VMEM((IDX_PER_CHUNK, 128), jnp.float32),
        pltpu.VMEM((2 * CHUNK, HID), jnp.float32),
        pltpu.VMEM((HID,), jnp.float32),
        pltpu.SemaphoreType.DMA,
        pltpu.SemaphoreType.DMA,
    ],
    compiler_params=pltpu.CompilerParams(use_tc_tiling_on_sc=True),
)(_encoder_body)


def _transpose_body(lut_t_ref, out_ref):
    for w in range(TCW):
        x = jnp.concatenate(
            [lut_t_ref[:, pl.ds(w * RBLK + u * 128, 128)]
             for u in range(SUPER)], axis=0)
        out_ref[pl.ds(w * 128, 128), :] = jnp.transpose(x)


_transpose_tc = pl.pallas_call(
    _transpose_body,
    out_shape=jax.ShapeDtypeStruct((NSUPER, SUPER * HID), jnp.float32),
    grid=(NTSTEP,),
    in_specs=[pl.BlockSpec((HID, TCBLK), lambda j: (0, j))],
    out_specs=pl.BlockSpec((TCBLK // SUPER, SUPER * HID), lambda j: (j, 0)),
)


def kernel(inp, lut, bias):
    inp_flat = inp.reshape(-1).astype(jnp.int32)
    lut_wide = _transpose_tc(lut.T)
    return _encoder(inp_flat, lut_wide, bias)
